# baseline (speedup 1.0000x reference)
import functools

import jax
import jax.numpy as jnp
from jax import lax
from jax.experimental import pallas as pl
from jax.experimental.pallas import tpu as pltpu
from jax.experimental.pallas import tpu_sc as plsc

NC = 2
NS = 16
NW = NC * NS
L = 16


def _make_argmin_body(has_xd):
    def body(*refs):
        if has_xd:
            (flat_ref, xd_ref, cb_ref, cn_ref,
             idx_ref, flat_out, s_out, f_ref, s_ref, sval_ref, sidx_ref) = refs
        else:
            (flat_ref, cb_ref, cn_ref,
             idx_ref, f_ref, s_ref, sval_ref, sidx_ref) = refs
        kb = pl.program_id(1)
        tm = f_ref.shape[0]
        tk = cb_ref.shape[0]

        @pl.when(kb == 0)
        def _init():
            if has_xd:
                flat = flat_ref[...] - xd_ref[...]
                f_ref[...] = flat
                flat_out[...] = flat
            else:
                flat = flat_ref[...]
                f_ref[...] = flat
            s = jnp.sum(flat ** 2, axis=1, keepdims=True)
            s_ref[...] = s
            if has_xd:
                s_out[...] = s
            sval_ref[...] = jnp.full((tm, 1), jnp.inf, jnp.float32)
            sidx_ref[...] = jnp.zeros((tm, 1), jnp.float32)

        mm = lax.dot_general(
            f_ref[...], cb_ref[...], (((1,), (1,)), ((), ())),
            preferred_element_type=jnp.float32)
        dist = s_ref[...] - 2.0 * mm + cn_ref[...]
        m = jnp.min(dist, axis=1, keepdims=True)
        iota = lax.broadcasted_iota(jnp.int32, (tm, tk), 1).astype(jnp.float32)
        tile_idx = (jnp.min(jnp.where(dist == m, iota, jnp.float32(2**30)),
                            axis=1, keepdims=True)
                    + jnp.float32(kb * tk))
        better = m < sval_ref[...]
        sval_ref[...] = jnp.where(better, m, sval_ref[...])
        sidx_ref[...] = jnp.where(better, tile_idx, sidx_ref[...])

        @pl.when(kb == pl.num_programs(1) - 1)
        def _fin():
            idx_ref[...] = sidx_ref[...].astype(jnp.int32)

    return body


def _argmin_call(flat, xd, cb, cn, tm, tk):
    n, d = flat.shape
    k = cb.shape[0]
    grid = (n // tm, k // tk)
    has_xd = xd is not None
    in_specs = [pl.BlockSpec((tm, d), lambda tb, kb: (tb, 0))]
    args = [flat]
    if has_xd:
        in_specs.append(pl.BlockSpec((tm, d), lambda tb, kb: (tb, 0)))
        args.append(xd)
    in_specs += [
        pl.BlockSpec((tk, d), lambda tb, kb: (kb, 0)),
        pl.BlockSpec((1, tk), lambda tb, kb: (0, kb)),
    ]
    args += [cb, cn.reshape(1, k)]
    out_specs = [pl.BlockSpec((tm, 1), lambda tb, kb: (tb, 0))]
    out_shape = [jax.ShapeDtypeStruct((n, 1), jnp.int32)]
    if has_xd:
        out_specs += [pl.BlockSpec((tm, d), lambda tb, kb: (tb, 0)),
                      pl.BlockSpec((tm, 1), lambda tb, kb: (tb, 0))]
        out_shape += [jax.ShapeDtypeStruct((n, d), jnp.float32),
                      jax.ShapeDtypeStruct((n, 1), jnp.float32)]
    out = pl.pallas_call(
        _make_argmin_body(has_xd),
        grid=grid,
        in_specs=in_specs,
        out_specs=out_specs,
        out_shape=out_shape,
        scratch_shapes=[
            pltpu.VMEM((tm, d), jnp.float32),
            pltpu.VMEM((tm, 1), jnp.float32),
            pltpu.VMEM((tm, 1), jnp.float32),
            pltpu.VMEM((tm, 1), jnp.float32),
        ],
        compiler_params=pltpu.CompilerParams(
            dimension_semantics=("arbitrary", "arbitrary")),
    )(*args)
    return tuple(out)


@functools.lru_cache(maxsize=None)
def _make_sc_gather(n, d, k):
    perw = n // NW
    ch = 96
    nch = perw // ch
    assert perw % ch == 0

    def body(cb_hbm, idx_hbm, xd_hbm, idx_v, rows0, rows1, sem0, sem1):
        wid = lax.axis_index("s") * NC + lax.axis_index("c")
        base = wid * perw
        pltpu.sync_copy(idx_hbm.at[pl.ds(base, perw)], idx_v)
        rows = (rows0, rows1)
        sems = (sem0, sem1)
        descs = [None, None]
        for c in range(nch):
            bb = c % 2
            descs[bb] = pltpu.async_copy(
                cb_hbm.at[idx_v.at[pl.ds(c * ch, ch)]], rows[bb], sems[bb])
            if c > 0:
                descs[1 - bb].wait()
                pltpu.sync_copy(rows[1 - bb],
                                xd_hbm.at[pl.ds(base + (c - 1) * ch, ch)])
        descs[(nch - 1) % 2].wait()
        pltpu.sync_copy(rows[(nch - 1) % 2],
                        xd_hbm.at[pl.ds(base + (nch - 1) * ch, ch)])

    mesh = plsc.VectorSubcoreMesh(core_axis_name="c", subcore_axis_name="s")
    return pl.kernel(
        body,
        out_type=jax.ShapeDtypeStruct((n, d), jnp.float32),
        mesh=mesh,
        scratch_types=(pltpu.VMEM((perw,), jnp.int32),
                       pltpu.VMEM((ch, d), jnp.float32),
                       pltpu.VMEM((ch, d), jnp.float32),
                       pltpu.SemaphoreType.DMA,
                       pltpu.SemaphoreType.DMA))


@functools.lru_cache(maxsize=None)
def _make_sc_final(n, d, k):
    perw = n // NW
    ch = 96
    nch = perw // ch
    assert perw % ch == 0

    def body(cb_hbm, idx_hbm, r_hbm, x0_hbm, qout_hbm, loss_hbm,
             idx_v, rows_v, r_v, x_v, acc_v, sem):
        wid = lax.axis_index("s") * NC + lax.axis_index("c")
        base = wid * perw
        zeros = jnp.zeros((L,), jnp.float32)
        acc_v[...] = zeros
        for c in range(nch):
            cbase = base + c * ch
            pltpu.sync_copy(idx_hbm.at[pl.ds(cbase, ch)], idx_v)
            pltpu.async_copy(cb_hbm.at[idx_v], rows_v, sem).wait()
            pltpu.sync_copy(r_hbm.at[pl.ds(cbase, ch)], r_v)
            pltpu.sync_copy(x0_hbm.at[pl.ds(cbase, ch)], x_v)

            def tbody(t, acc):
                for j in range(d // L):
                    sl = pl.ds(j * L, L)
                    dlt = r_v[t, sl] - rows_v[t, sl]
                    x_v[t, sl] = x_v[t, sl] - dlt
                    acc = acc + dlt * dlt
                return acc
            acc_v[...] = lax.fori_loop(0, ch, tbody, acc_v[...])
            pltpu.sync_copy(x_v, qout_hbm.at[pl.ds(cbase, ch)])
        pltpu.sync_copy(acc_v, loss_hbm.at[wid])

    mesh = plsc.VectorSubcoreMesh(core_axis_name="c", subcore_axis_name="s")
    return pl.kernel(
        body,
        out_type=(jax.ShapeDtypeStruct((n, d), jnp.float32),
                  jax.ShapeDtypeStruct((NW, L), jnp.float32)),
        mesh=mesh,
        scratch_types=(pltpu.VMEM((ch,), jnp.int32),
                       pltpu.VMEM((ch, d), jnp.float32),
                       pltpu.VMEM((ch, d), jnp.float32),
                       pltpu.VMEM((ch, d), jnp.float32),
                       pltpu.VMEM((L,), jnp.float32),
                       pltpu.SemaphoreType.DMA))


@functools.lru_cache(maxsize=None)
def _make_sc_hist(n, k):
    perw = n // NW

    def body(idx_hbm, counts_hbm, idx_v, counts_v):
        wid = lax.axis_index("s") * NC + lax.axis_index("c")
        base = wid * perw
        zeros = jnp.zeros((L,), jnp.float32)
        ones = jnp.ones((L,), jnp.float32)
        lane = lax.iota(jnp.int32, L)
        one0 = jnp.where(lane == 0, ones, zeros)

        def zbody(i, c):
            counts_v[pl.ds(i * L, L)] = zeros
            return c
        lax.fori_loop(0, k // L + 1, zbody, 0)
        pltpu.sync_copy(idx_hbm.at[pl.ds(base, perw)], idx_v)
        for v in range(perw // L):
            iv = idx_v[pl.ds(v * L, L)]
            for j in range(L):
                si = iv[j]
                cv = counts_v[pl.ds(si, L)]
                counts_v[pl.ds(si, L)] = cv + one0
        pltpu.sync_copy(counts_v.at[pl.ds(0, k)], counts_hbm.at[wid])

    mesh = plsc.VectorSubcoreMesh(core_axis_name="c", subcore_axis_name="s")
    return pl.kernel(
        body,
        out_type=jax.ShapeDtypeStruct((NW, k), jnp.float32),
        mesh=mesh,
        scratch_types=(pltpu.VMEM((perw,), jnp.int32),
                       pltpu.VMEM((k + L,), jnp.float32)))


def kernel(x, codebooks):
    b, d, t = x.shape
    nq, k, _ = codebooks.shape
    n = b * t
    cn = jnp.sum(codebooks ** 2, axis=-1)
    flat0 = jnp.transpose(x, (0, 2, 1)).reshape(n, d)

    tm = 2304 if n % 2304 == 0 else n
    tk = 2048 if k % 2048 == 0 else k

    flat = flat0
    xd = None
    idxs, s_list, counts_list = [], [], []
    losspart = None
    qout_flat = None
    for q in range(nq):
        outs = _argmin_call(flat, xd, codebooks[q], cn[q], tm, tk)
        if q == 0:
            (idx,) = outs
        else:
            idx, flat, s = outs
            s_list.append(s)
        idx1 = idx[:, 0]
        idxs.append(idx1.reshape(b, t))
        if q < nq - 1:
            xd = _make_sc_gather(n, d, k)(codebooks[q], idx1)
        else:
            qout_flat, losspart = _make_sc_final(n, d, k)(
                codebooks[q], idx1, flat, flat0)
        counts_list.append(_make_sc_hist(n, k)(idx1))

    losses = [jnp.sum(s) / (n * d) for s in s_list]
    losses.append(jnp.sum(losspart) / (n * d))
    perps = []
    for counts in counts_list:
        prob = jnp.sum(counts, axis=0) / n
        perps.append(jnp.exp(-jnp.sum(prob * jnp.log(prob + 1e-7))))

    all_indices = jnp.stack(idxs, axis=-1)
    vq_loss = sum(losses) / nq
    perplexity = sum(perps) / nq
    quantized_out = jnp.transpose(qout_flat.reshape(b, t, d), (0, 2, 1))
    return quantized_out, all_indices, vq_loss, perplexity
